# baseline (speedup 1.0000x reference)
import jax
import jax.numpy as jnp
from jax import lax
from jax.experimental import pallas as pl
from jax.experimental.pallas import tpu as pltpu
from jax.experimental.pallas import tpu_sc as plsc

B = 16384
D = 32
NC = 2
NS = 16
NW = NC * NS
BPW = B // NW
EPW = BPW * D


def _gather_body(f0, f1, f2, f3, e0, e1, e2, e3, out_hbm, eidx_v, vals_v, sem):
    wid = lax.axis_index("s") * NC + lax.axis_index("c")
    base = wid * EPW
    for t, (fh, eh) in enumerate(((f0, e0), (f1, e1), (f2, e2), (f3, e3))):
        pltpu.sync_copy(eh.at[pl.ds(base, EPW)], eidx_v)
        pltpu.async_copy(fh.at[eidx_v], vals_v, sem).wait()
        pltpu.sync_copy(vals_v, out_hbm.at[pl.ds((t * B * D) + base, EPW)])


def kernel(room_id, hotel, room_type, room_name,
           room_table, hotel_table, room_type_table, room_name_table):
    mesh = plsc.VectorSubcoreMesh(core_axis_name="c", subcore_axis_name="s")
    gather = pl.kernel(
        _gather_body,
        out_type=jax.ShapeDtypeStruct((4 * B * D,), jnp.float32),
        mesh=mesh,
        scratch_types=[
            pltpu.VMEM((EPW,), jnp.int32),
            pltpu.VMEM((EPW,), jnp.float32),
            pltpu.SemaphoreType.DMA,
        ],
    )
    lane = jnp.arange(D, dtype=jnp.int32)
    eidx = [
        (idx.astype(jnp.int32)[:, None] * D + lane[None, :]).reshape(-1)
        for idx in (room_id, hotel, room_type, room_name)
    ]
    flats = [t.reshape(-1) for t in
             (room_table, hotel_table, room_type_table, room_name_table)]
    out = gather(*flats, *eidx)
    return out.reshape(4, B, D).transpose(1, 0, 2).reshape(B, 4 * D)
